# baseline (speedup 1.0000x reference)
import functools

import jax
import jax.numpy as jnp
from jax import lax
from jax.experimental import pallas as pl
from jax.experimental.pallas import tpu as pltpu
from jax.experimental.pallas import tpu_sc as plsc

D = 128
NUM_CORES = 2
NUM_SUBCORES = 16
NW = NUM_CORES * NUM_SUBCORES
IDX_W = 128
K = 1
NBUF = 4


def _make_gather(n_rows_total):
    rows_per_w = n_rows_total // NW
    n_steps = rows_per_w // K
    mesh = plsc.VectorSubcoreMesh(core_axis_name="c", subcore_axis_name="s")

    @functools.partial(
        pl.kernel,
        mesh=mesh,
        out_type=jax.ShapeDtypeStruct((n_rows_total * IDX_W, D), jnp.float32),
        scratch_types=[
            pltpu.VMEM((rows_per_w, IDX_W), jnp.int32),
            pltpu.VMEM((NBUF, K * IDX_W, D), jnp.float32),
        ] + [pltpu.SemaphoreType.DMA] * (2 * NBUF),
    )
    def gather_kernel(idx_hbm, table_hbm, out_hbm, idx_v, rows_v, *sems):
        wid = lax.axis_index("s") * NUM_CORES + lax.axis_index("c")
        row0 = wid * rows_per_w
        gsem = list(sems[:NBUF])
        osem = list(sems[NBUF:])

        pltpu.sync_copy(idx_hbm.at[pl.ds(row0, rows_per_w)], idx_v)

        def fire_gathers(g, b):
            for j in range(K):
                pltpu.async_copy(
                    table_hbm.at[idx_v.at[g * K + j]],
                    rows_v.at[b].at[pl.ds(j * IDX_W, IDX_W)],
                    gsem[b],
                )

        def wait_gathers(b):
            for j in range(K):
                pltpu.make_async_copy(
                    table_hbm.at[idx_v.at[0]],
                    rows_v.at[b].at[pl.ds(j * IDX_W, IDX_W)],
                    gsem[b],
                ).wait()

        def fire_out(g, b):
            pltpu.async_copy(
                rows_v.at[b],
                out_hbm.at[pl.ds((row0 + g * K) * IDX_W, K * IDX_W)],
                osem[b],
            )

        def wait_out(b):
            pltpu.make_async_copy(
                rows_v.at[b],
                out_hbm.at[pl.ds(0, K * IDX_W)],
                osem[b],
            ).wait()

        for f in range(NBUF - 1):
            fire_gathers(f, f)

        @pl.loop(0, n_steps, step=NBUF)
        def _(g_base):
            for b in range(NBUF):
                g = g_base + b
                f = g + NBUF - 1
                fb = (b + NBUF - 1) % NBUF

                @pl.when(jnp.logical_and(f >= NBUF, f < n_steps))
                def _():
                    wait_out(fb)

                @pl.when(f < n_steps)
                def _():
                    fire_gathers(f, fb)

                wait_gathers(b)
                fire_out(g, b)

        for b in range(NBUF):
            wait_out(b)

    return gather_kernel


def kernel(x, table):
    B, L = x.shape
    n = B * L
    idx = x.reshape(n // IDX_W, IDX_W)
    out = _make_gather(n // IDX_W)(idx, table)
    return out.reshape(B, L, D)


# speedup vs baseline: 1.0030x; 1.0030x over previous
import functools

import jax
import jax.numpy as jnp
from jax import lax
from jax.experimental import pallas as pl
from jax.experimental.pallas import tpu as pltpu
from jax.experimental.pallas import tpu_sc as plsc

D = 128
NUM_CORES = 2
NUM_SUBCORES = 16
NW = NUM_CORES * NUM_SUBCORES
IDX_W = 128
K = 2
NBUF = 3


def _make_gather(n_rows_total):
    rows_per_w = n_rows_total // NW
    n_steps = rows_per_w // K
    mesh = plsc.VectorSubcoreMesh(core_axis_name="c", subcore_axis_name="s")

    @functools.partial(
        pl.kernel,
        mesh=mesh,
        out_type=jax.ShapeDtypeStruct((n_rows_total * IDX_W, D), jnp.float32),
        scratch_types=[
            pltpu.VMEM((rows_per_w, IDX_W), jnp.int32),
            pltpu.VMEM((NBUF, K * IDX_W, D), jnp.float32),
        ] + [pltpu.SemaphoreType.DMA] * (2 * NBUF),
    )
    def gather_kernel(idx_hbm, table_hbm, out_hbm, idx_v, rows_v, *sems):
        wid = lax.axis_index("s") * NUM_CORES + lax.axis_index("c")
        row0 = wid * rows_per_w
        gsem = list(sems[:NBUF])
        osem = list(sems[NBUF:])

        pltpu.sync_copy(idx_hbm.at[pl.ds(row0, rows_per_w)], idx_v)

        def fire_gathers(g, b):
            for j in range(K):
                pltpu.async_copy(
                    table_hbm.at[idx_v.at[g * K + j]],
                    rows_v.at[b].at[pl.ds(j * IDX_W, IDX_W)],
                    gsem[b],
                )

        def wait_gathers(b):
            for j in range(K):
                pltpu.make_async_copy(
                    table_hbm.at[idx_v.at[0]],
                    rows_v.at[b].at[pl.ds(j * IDX_W, IDX_W)],
                    gsem[b],
                ).wait()

        def fire_out(g, b):
            pltpu.async_copy(
                rows_v.at[b],
                out_hbm.at[pl.ds((row0 + g * K) * IDX_W, K * IDX_W)],
                osem[b],
            )

        def wait_out(b):
            pltpu.make_async_copy(
                rows_v.at[b],
                out_hbm.at[pl.ds(0, K * IDX_W)],
                osem[b],
            ).wait()

        for f in range(NBUF - 1):
            fire_gathers(f, f)

        @pl.loop(0, n_steps, step=NBUF)
        def _(g_base):
            for b in range(NBUF):
                g = g_base + b
                f = g + NBUF - 1
                fb = (b + NBUF - 1) % NBUF

                @pl.when(jnp.logical_and(f >= NBUF, f < n_steps))
                def _():
                    wait_out(fb)

                @pl.when(f < n_steps)
                def _():
                    fire_gathers(f, fb)

                @pl.when(g < n_steps)
                def _():
                    wait_gathers(b)
                    fire_out(g, b)

        for b in range(NBUF):
            wait_out(b)

    return gather_kernel


def kernel(x, table):
    B, L = x.shape
    n = B * L
    idx = x.reshape(n // IDX_W, IDX_W)
    out = _make_gather(n // IDX_W)(idx, table)
    return out.reshape(B, L, D)
